# baseline (speedup 1.0000x reference)
import dataclasses
import functools

import jax
import jax.numpy as jnp
import numpy as np
from jax import lax
from jax.experimental import pallas as pl
from jax.experimental.pallas import tpu as pltpu
from jax.experimental.pallas import tpu_sc as plsc

NC, NS, L = 2, 16, 16
NW = NC * NS

_GATHER_DNUMS = lax.GatherDimensionNumbers(
    offset_dims=(), collapsed_slice_dims=(0,), start_index_map=(0,)
)


def _lane_perm(v, perm_col):
    return lax.gather(
        v,
        perm_col,
        _GATHER_DNUMS,
        slice_sizes=(1,),
        mode=lax.GatherScatterMode.PROMISE_IN_BOUNDS,
    )


@functools.lru_cache(maxsize=None)
def _build(E, H, R, W):
    EW = E // NW
    C = EW // W
    mesh = plsc.VectorSubcoreMesh(
        core_axis_name="c", subcore_axis_name="s", num_cores=NC, num_subcores=NS
    )
    cp = pltpu.CompilerParams()
    if "needs_layout_passes" in pltpu.CompilerParams.__dataclass_fields__:
        cp = dataclasses.replace(cp, needs_layout_passes=False)

    @functools.partial(
        pl.kernel,
        compiler_params=cp,
        out_type=jax.ShapeDtypeStruct((NW, C, W), jnp.float32),
        mesh=mesh,
        scratch_types=[
            pltpu.VMEM((C, W), jnp.int32),
            pltpu.VMEM((C, W), jnp.int32),
            pltpu.VMEM((C, W), jnp.int32),
            pltpu.VMEM((W, H), jnp.float32),
            pltpu.VMEM((W, H), jnp.float32),
            pltpu.VMEM((W, H), jnp.float32),
            pltpu.VMEM((W, H), jnp.float32),
            pltpu.VMEM((R, H), jnp.float32),
            pltpu.VMEM((C, W), jnp.float32),
            pltpu.SemaphoreType.DMA,
            pltpu.SemaphoreType.DMA,
        ],
    )
    def k(z_hbm, src_hbm, dst_hbm, typ_hbm, rel_hbm, out_hbm,
          src_v, dst_v, typ_v, srowsA, drowsA, srowsB, drowsB, rel_v, out_v,
          semA, semB):
        wid = lax.axis_index("s") * NC + lax.axis_index("c")
        pltpu.sync_copy(src_hbm.at[wid], src_v)
        pltpu.sync_copy(dst_hbm.at[wid], dst_v)
        pltpu.sync_copy(typ_hbm.at[wid], typ_v)
        pltpu.sync_copy(rel_hbm, rel_v)
        lanes = lax.iota(jnp.int32, L)
        m_last = lanes == (L - 1)

        def start(kk, srows, drows, sem):
            pltpu.async_copy(z_hbm.at[src_v.at[kk]], srows, sem)
            pltpu.async_copy(z_hbm.at[dst_v.at[kk]], drows, sem)

        def drain(srows, drows, sem):
            pltpu.make_async_copy(z_hbm.at[src_v.at[0]], srows, sem).wait()
            pltpu.make_async_copy(z_hbm.at[dst_v.at[0]], drows, sem).wait()

        def compute(kk, srows, drows):
            @pl.loop(0, W // L)
            def _group(g):
                tv = typ_v[kk, pl.ds(g * L, L)]
                colb = jnp.broadcast_to(g * L, (L,))
                for u in range(L):
                    e = g * L + u
                    t = tv[u]
                    a0 = a1 = None
                    for q in range(H // L):
                        s = srows[e, pl.ds(q * L, L)]
                        d = drows[e, pl.ds(q * L, L)]
                        r = rel_v[t, pl.ds(q * L, L)]
                        p = s * d * r
                        if q % 2 == 0:
                            a0 = p if a0 is None else a0 + p
                        else:
                            a1 = p if a1 is None else a1 + p
                    c = plsc.cumsum(a0 + a1)
                    col = colb + u
                    plsc.store_scatter(out_v.at[kk], [col], c, mask=m_last)

        start(0, srowsA, drowsA, semA)

        @pl.loop(0, C)
        def _chunk(kk):
            @pl.when(kk % 2 == 0)
            def _even():
                drain(srowsA, drowsA, semA)

                @pl.when(kk + 1 < C)
                def _():
                    start(kk + 1, srowsB, drowsB, semB)

                compute(kk, srowsA, drowsA)

            @pl.when(kk % 2 == 1)
            def _odd():
                drain(srowsB, drowsB, semB)

                @pl.when(kk + 1 < C)
                def _():
                    start(kk + 1, srowsA, drowsA, semA)

                compute(kk, srowsB, drowsB)

        pltpu.sync_copy(out_v, out_hbm.at[wid])

    return k


def kernel(z, edge_index, edge_type, rel_emb):
    E = edge_type.shape[0]
    H = z.shape[1]
    R = rel_emb.shape[0]
    W = 80
    C = E // (NW * W)
    src = edge_index[0].astype(jnp.int32).reshape(NW, C, W)
    dst = edge_index[1].astype(jnp.int32).reshape(NW, C, W)
    typ = edge_type.astype(jnp.int32).reshape(NW, C, W)
    out = _build(E, H, R, W)(z, src, dst, typ, rel_emb)
    return out.reshape(E)


# speedup vs baseline: 1.0434x; 1.0434x over previous
import dataclasses
import functools

import jax
import jax.numpy as jnp
from jax import lax
from jax.experimental import pallas as pl
from jax.experimental.pallas import tpu as pltpu
from jax.experimental.pallas import tpu_sc as plsc

NC, NS, L = 2, 16, 16
NW = NC * NS


@functools.lru_cache(maxsize=None)
def _build(E, H, R, W):
    EW = E // NW
    C = EW // W
    HW = H // 2
    mesh = plsc.VectorSubcoreMesh(
        core_axis_name="c", subcore_axis_name="s", num_cores=NC, num_subcores=NS
    )
    cp = pltpu.CompilerParams()
    if "needs_layout_passes" in pltpu.CompilerParams.__dataclass_fields__:
        cp = dataclasses.replace(cp, needs_layout_passes=False)

    @functools.partial(
        pl.kernel,
        compiler_params=cp,
        out_type=jax.ShapeDtypeStruct((NW, C, W), jnp.float32),
        mesh=mesh,
        scratch_types=[
            pltpu.VMEM((C, W), jnp.int32),
            pltpu.VMEM((C, W), jnp.int32),
            pltpu.VMEM((C, W), jnp.int32),
            pltpu.VMEM((C, W), jnp.int32),
            pltpu.VMEM((W, H), jnp.int32),
            pltpu.VMEM((W, H), jnp.int32),
            pltpu.VMEM((W, H), jnp.int32),
            pltpu.VMEM((W, H), jnp.int32),
            pltpu.VMEM((R, HW), jnp.int32),
            pltpu.VMEM((C, W), jnp.float32),
            pltpu.SemaphoreType.DMA,
            pltpu.SemaphoreType.DMA,
        ],
    )
    def k(zp_hbm, src_hbm, dst_hbm, typ_hbm, rel_hbm, out_hbm,
          src_v, dst_v, typ_v, par_v,
          srowsA, drowsA, srowsB, drowsB, rel_v, out_v, semA, semB):
        wid = lax.axis_index("s") * NC + lax.axis_index("c")
        pltpu.sync_copy(src_hbm.at[wid], src_v)
        pltpu.sync_copy(dst_hbm.at[wid], dst_v)
        pltpu.sync_copy(typ_hbm.at[wid], typ_v)
        pltpu.sync_copy(rel_hbm, rel_v)
        lanes = lax.iota(jnp.int32, L)
        m_last = lanes == (L - 1)

        @pl.loop(0, C)
        def _prep(kk):
            @pl.loop(0, W // L)
            def _prep_g(g):
                sl = pl.ds(g * L, L)
                sv = src_v[kk, sl]
                dv = dst_v[kk, sl]
                par_v[kk, sl] = ((sv & 1) * HW) | ((dv & 1) << 16) * HW
                src_v[kk, sl] = sv >> 1
                dst_v[kk, sl] = dv >> 1

        def start(kk, srows, drows, sem):
            pltpu.async_copy(zp_hbm.at[src_v.at[kk]], srows, sem)
            pltpu.async_copy(zp_hbm.at[dst_v.at[kk]], drows, sem)

        def drain(srows, drows, sem):
            pltpu.make_async_copy(zp_hbm.at[src_v.at[0]], srows, sem).wait()
            pltpu.make_async_copy(zp_hbm.at[dst_v.at[0]], drows, sem).wait()

        def compute(kk, srows, drows):
            @pl.loop(0, W // L)
            def _group(g):
                sl = pl.ds(g * L, L)
                tv = typ_v[kk, sl]
                pv = par_v[kk, sl]
                colb = jnp.broadcast_to(g * L, (L,))
                fmt = plsc.PackFormat.INTERLEAVED
                for u in range(L):
                    e = g * L + u
                    t = tv[u]
                    pw = pv[u]
                    ps = pw & 0xFFFF
                    pd = lax.shift_right_logical(pw, 16)
                    a0 = a1 = None
                    for q in range(HW // L):
                        sab = plsc.bitcast(
                            srows[e, pl.ds(ps + q * L, L)], jnp.bfloat16)
                        dab = plsc.bitcast(
                            drows[e, pl.ds(pd + q * L, L)], jnp.bfloat16)
                        rab = plsc.bitcast(
                            rel_v[t, pl.ds(q * L, L)], jnp.bfloat16)
                        prod = sab * dab * rab
                        p0, p1 = plsc.unpack(prod, format=fmt)
                        a0 = p0 if a0 is None else a0 + p0
                        a1 = p1 if a1 is None else a1 + p1
                    c = plsc.cumsum(a0 + a1)
                    col = colb + u
                    plsc.store_scatter(out_v.at[kk], [col], c, mask=m_last)

        start(0, srowsA, drowsA, semA)

        @pl.loop(0, C)
        def _chunk(kk):
            @pl.when(kk % 2 == 0)
            def _even():
                drain(srowsA, drowsA, semA)

                @pl.when(kk + 1 < C)
                def _():
                    start(kk + 1, srowsB, drowsB, semB)

                compute(kk, srowsA, drowsA)

            @pl.when(kk % 2 == 1)
            def _odd():
                drain(srowsB, drowsB, semB)

                @pl.when(kk + 1 < C)
                def _():
                    start(kk + 1, srowsA, drowsA, semA)

                compute(kk, srowsB, drowsB)

        pltpu.sync_copy(out_v, out_hbm.at[wid])

    return k


def kernel(z, edge_index, edge_type, rel_emb):
    E = edge_type.shape[0]
    H = z.shape[1]
    R = rel_emb.shape[0]
    W = 80
    C = E // (NW * W)
    src = edge_index[0].astype(jnp.int32).reshape(NW, C, W)
    dst = edge_index[1].astype(jnp.int32).reshape(NW, C, W)
    typ = edge_type.astype(jnp.int32).reshape(NW, C, W)
    n = z.shape[0]
    z_pair = lax.bitcast_convert_type(
        z.astype(jnp.bfloat16).reshape(n, H // 2, 2), jnp.int32
    ).reshape(n // 2, H)
    rel_i32 = lax.bitcast_convert_type(
        rel_emb.astype(jnp.bfloat16).reshape(R, H // 2, 2), jnp.int32
    )
    out = _build(E, H, R, W)(z_pair, src, dst, typ, rel_i32)
    return out.reshape(E)


# speedup vs baseline: 1.0477x; 1.0042x over previous
import dataclasses
import functools

import jax
import jax.numpy as jnp
from jax import lax
from jax.experimental import pallas as pl
from jax.experimental.pallas import tpu as pltpu
from jax.experimental.pallas import tpu_sc as plsc

NC, NS, L = 2, 16, 16
NW = NC * NS


@functools.lru_cache(maxsize=None)
def _build(E, H, R, W):
    EW = E // NW
    C = EW // W
    HW = H // 2
    mesh = plsc.VectorSubcoreMesh(
        core_axis_name="c", subcore_axis_name="s", num_cores=NC, num_subcores=NS
    )
    cp = pltpu.CompilerParams()
    if "needs_layout_passes" in pltpu.CompilerParams.__dataclass_fields__:
        cp = dataclasses.replace(cp, needs_layout_passes=False)

    @functools.partial(
        pl.kernel,
        compiler_params=cp,
        out_type=jax.ShapeDtypeStruct((NW, C, W), jnp.float32),
        mesh=mesh,
        scratch_types=[
            pltpu.VMEM((C, W), jnp.int32),
            pltpu.VMEM((C, W), jnp.int32),
            pltpu.VMEM((C, W), jnp.int32),
            pltpu.VMEM((W, H), jnp.int32),
            pltpu.VMEM((W, H), jnp.int32),
            pltpu.VMEM((W, H), jnp.int32),
            pltpu.VMEM((W, H), jnp.int32),
            pltpu.VMEM((R, HW), jnp.int32),
            pltpu.VMEM((C, W), jnp.float32),
            pltpu.SemaphoreType.DMA,
            pltpu.SemaphoreType.DMA,
        ],
    )
    def k(zp_hbm, src_hbm, dst_hbm, typ_hbm, rel_hbm, out_hbm,
          src_v, dst_v, typ_v,
          srowsA, drowsA, srowsB, drowsB, rel_v, out_v, semA, semB):
        wid = lax.axis_index("s") * NC + lax.axis_index("c")
        pltpu.sync_copy(src_hbm.at[wid], src_v)
        pltpu.sync_copy(dst_hbm.at[wid], dst_v)
        pltpu.sync_copy(typ_hbm.at[wid], typ_v)
        pltpu.sync_copy(rel_hbm, rel_v)
        lanes = lax.iota(jnp.int32, L)
        m_last = lanes == (L - 1)

        def start(kk, srows, drows, sem):
            pltpu.async_copy(zp_hbm.at[src_v.at[kk]], srows, sem)
            pltpu.async_copy(zp_hbm.at[dst_v.at[kk]], drows, sem)

        def drain(srows, drows, sem):
            pltpu.make_async_copy(zp_hbm.at[src_v.at[0]], srows, sem).wait()
            pltpu.make_async_copy(zp_hbm.at[dst_v.at[0]], drows, sem).wait()

        def compute(kk, srows, drows):
            @pl.loop(0, W // L)
            def _group(g):
                sl = pl.ds(g * L, L)
                tv = typ_v[kk, sl]
                colb = jnp.broadcast_to(g * L, (L,))
                fmt = plsc.PackFormat.INTERLEAVED
                for u in range(L):
                    e = g * L + u
                    t = tv[u]
                    a0 = a1 = None
                    for q in range(HW // L):
                        sab = plsc.bitcast(
                            srows[e, pl.ds(q * L, L)], jnp.bfloat16)
                        dab = plsc.bitcast(
                            drows[e, pl.ds(q * L, L)], jnp.bfloat16)
                        rab = plsc.bitcast(
                            rel_v[t, pl.ds(q * L, L)], jnp.bfloat16)
                        prod = sab * dab * rab
                        p0, p1 = plsc.unpack(prod, format=fmt)
                        a0 = p0 if a0 is None else a0 + p0
                        a1 = p1 if a1 is None else a1 + p1
                    c = plsc.cumsum(a0 + a1)
                    col = colb + u
                    plsc.store_scatter(out_v.at[kk], [col], c, mask=m_last)

        start(0, srowsA, drowsA, semA)

        @pl.loop(0, C)
        def _chunk(kk):
            @pl.when(kk % 2 == 0)
            def _even():
                drain(srowsA, drowsA, semA)

                @pl.when(kk + 1 < C)
                def _():
                    start(kk + 1, srowsB, drowsB, semB)

                compute(kk, srowsA, drowsA)

            @pl.when(kk % 2 == 1)
            def _odd():
                drain(srowsB, drowsB, semB)

                @pl.when(kk + 1 < C)
                def _():
                    start(kk + 1, srowsA, drowsA, semA)

                compute(kk, srowsB, drowsB)

        pltpu.sync_copy(out_v, out_hbm.at[wid])

    return k


def kernel(z, edge_index, edge_type, rel_emb):
    E = edge_type.shape[0]
    H = z.shape[1]
    R = rel_emb.shape[0]
    W = 80
    C = E // (NW * W)
    src = edge_index[0].astype(jnp.int32).reshape(NW, C, W)
    dst = edge_index[1].astype(jnp.int32).reshape(NW, C, W)
    typ = edge_type.astype(jnp.int32).reshape(NW, C, W)
    n = z.shape[0]
    zw = lax.bitcast_convert_type(
        z.astype(jnp.bfloat16).reshape(n, H // 2, 2), jnp.int32
    )
    z_tab = jnp.concatenate([zw, zw], axis=1)
    rel_i32 = lax.bitcast_convert_type(
        rel_emb.astype(jnp.bfloat16).reshape(R, H // 2, 2), jnp.int32
    )
    out = _build(E, H, R, W)(z_tab, src, dst, typ, rel_i32)
    return out.reshape(E)
